# baseline (speedup 1.0000x reference)
from functools import partial

import jax
import jax.numpy as jnp
from jax import lax
from jax.experimental import pallas as pl
from jax.experimental.pallas import tpu as pltpu

_LANE = 128
_BF = jnp.bfloat16
_F32 = jnp.float32
_IMGS = 2

_SRC = ((1, -1), (0, 0), (1, 0), (0, 1))
_READERS = (((0, 3),), ((0, 1), (1, 2), (2, 3)), ((1, 0), (2, 1), (3, 2)),
            ((3, 0),))


def _ru(x, m):
    return (x + m - 1) // m * m


def _geom(h, w):
    w2 = w + 2
    l_out = h * w2
    l_out_p = _ru(max(l_out, _LANE), _LANE)
    l_in_p = _ru(2 * w2 + 2 + l_out_p, _LANE)
    return w2, l_out, l_out_p, l_in_p


def _decoder_kernel(xf, cw, cb, r0w3, r0b3, r0w1, r0b1, r1w3, r1b3, r1w1,
                    r1b1, u0l, u0b, wbig, bout, o_ref,
                    s1, s2a, s2b, s3, s4, hres, *,
                    c, ch, cu, w, w2, l_out, l_out_p, l_in_p):
    shift = w2 + 1
    tail = l_in_p - shift - l_out_p

    def mask(rows):
        lane = lax.broadcasted_iota(jnp.int32, (rows, l_out_p), 1)
        return (lane % w2 < w) & (lane < l_out)

    m_ch = mask(ch)
    m_cu = mask(cu)

    def padded(y, m):
        yb = jnp.where(m, jnp.maximum(y.astype(_BF), 0), 0)
        rows = yb.shape[0]
        return jnp.concatenate(
            [jnp.zeros((rows, shift), _BF), yb, jnp.zeros((rows, tail), _BF)],
            axis=1)

    def stack9(dst, img, f, rows):
        for t in range(9):
            off = (t // 3) * w2 + (t % 3)
            dst[img, t * rows:(t + 1) * rows, :] = f[:, off:off + l_out_p]

    for img in range(_IMGS):
        stack9(s1, img, xf[:, img * l_in_p:(img + 1) * l_in_p], c)
        h = jnp.dot(cw[...], s1[img], preferred_element_type=_F32) + cb[...]

        for w3, b3, w1, b1, s2 in ((r0w3, r0b3, r0w1, r0b1, s2a),
                                   (r1w3, r1b3, r1w1, r1b1, s2b)):
            hres[img] = h.astype(_BF)
            stack9(s2, img, padded(h, m_ch), ch)
            t3 = (jnp.dot(w3[...], s2[img], preferred_element_type=_F32)
                  + b3[...])
            t3 = jnp.maximum(t3.astype(_BF), 0)
            h = jnp.dot(w1[...], t3, preferred_element_type=_F32) + b1[...]
            h = h + hres[img].astype(_F32)

        stack9(s3, img, padded(h, m_ch), ch)
        for p in range(4):
            ry, rx = p // 2, p % 2
            r0 = (ry * 3 + rx) * ch
            acc = jnp.dot(u0l[2 * p], s3[img, r0:r0 + 2 * ch],
                          preferred_element_type=_F32)
            acc = acc + jnp.dot(u0l[2 * p + 1],
                                s3[img, r0 + 3 * ch:r0 + 5 * ch],
                                preferred_element_type=_F32)
            fp = padded(acc + u0b[...], m_cu)
            for ya in range(4):
                for xb in range(4):
                    if (_SRC[ya][0] * 2 + _SRC[xb][0]) != p:
                        continue
                    off = (1 + _SRC[ya][1]) * w2 + (1 + _SRC[xb][1])
                    s = ya * 4 + xb
                    s4[img, s * cu:(s + 1) * cu, :] = fp[:, off:off + l_out_p]

        o = (jnp.dot(wbig[...], s4[img], preferred_element_type=_F32)
             + bout[...])
        o_ref[:, img * l_out_p:(img + 1) * l_out_p] = o.astype(_BF)


def kernel(x, conv_w, conv_b, res0_w3, res0_b3, res0_w1, res0_b1,
           res1_w3, res1_b3, res1_w1, res1_b1, up0_w, up0_b, up1_w, up1_b):
    n, c, h, w = x.shape
    ch = conv_w.shape[0]
    crh = res0_w3.shape[0]
    cu = up0_w.shape[1]
    co = up1_w.shape[1]
    w2, l_out, l_out_p, l_in_p = _geom(h, w)

    xc = jnp.transpose(x, (1, 0, 2, 3)).astype(_BF)
    xp = jnp.pad(xc, ((0, 0), (0, 0), (1, 1), (1, 1)))
    xf = xp.reshape(c, n, (h + 2) * (w + 2))
    xf = jnp.pad(xf, ((0, 0), (0, 0), (0, l_in_p - (h + 2) * (w + 2))))
    xf = xf.reshape(c, n * l_in_p)

    cw = jnp.transpose(conv_w, (0, 2, 3, 1)).reshape(ch, 9 * c).astype(_BF)
    cb = conv_b.reshape(ch, 1)

    def res_w(w3, b3, w1, b1):
        return (jnp.transpose(w3, (0, 2, 3, 1)).reshape(crh, 9 * ch).astype(_BF),
                b3.reshape(crh, 1),
                w1.reshape(ch, crh).astype(_BF),
                b1.reshape(ch, 1))

    r0 = res_w(res0_w3, res0_b3, res0_w1, res0_b1)
    r1 = res_w(res1_w3, res1_b3, res1_w1, res1_b1)

    wt0 = jnp.transpose(up0_w, (1, 0, 2, 3))
    u0l = jnp.stack([
        jnp.concatenate([wt0[:, :, 3 - 2 * a - ry, 3 - rx],
                         wt0[:, :, 3 - 2 * a - ry, 1 - rx]], axis=1)
        for ry in (0, 1) for rx in (0, 1) for a in (0, 1)], axis=0).astype(_BF)
    u0b = up0_b.reshape(cu, 1)

    wt1 = jnp.transpose(up1_w, (1, 0, 2, 3))
    zero3 = jnp.zeros((co, cu), _F32)
    rows = []
    for qy in range(4):
        for qx in range(4):
            blocks = []
            for ya in range(4):
                khm = dict(_READERS[ya])
                for xb in range(4):
                    kwm = dict(_READERS[xb])
                    if qy in khm and qx in kwm:
                        blocks.append(wt1[:, :, khm[qy], kwm[qx]])
                    else:
                        blocks.append(zero3)
            rows.append(jnp.concatenate(blocks, axis=1))
    wbig = jnp.concatenate(rows, axis=0).astype(_BF)
    bout = jnp.tile(up1_b, 16).reshape(16 * co, 1)

    cparams = pltpu.CompilerParams(
        dimension_semantics=("parallel",),
        vmem_limit_bytes=64 * 1024 * 1024)
    flops_img = 2 * l_out * (9 * ch * c + 2 * (9 * crh * ch + ch * crh)
                             + 16 * cu * ch + 48 * 16 * cu)
    cost = pl.CostEstimate(flops=n * flops_img, transcendentals=0,
                           bytes_accessed=2 * n * c * l_in_p
                           + 2 * n * 16 * co * l_out_p)

    const = lambda i: (0, 0)
    const3 = lambda i: (0, 0, 0)
    out2d = pl.pallas_call(
        partial(_decoder_kernel, c=c, ch=ch, cu=cu, w=w, w2=w2, l_out=l_out,
                l_out_p=l_out_p, l_in_p=l_in_p),
        out_shape=jax.ShapeDtypeStruct((16 * co, n * l_out_p), _BF),
        grid=(n // _IMGS,),
        in_specs=[
            pl.BlockSpec((c, _IMGS * l_in_p), lambda i: (0, i)),
            pl.BlockSpec((ch, 9 * c), const),
            pl.BlockSpec((ch, 1), const),
            pl.BlockSpec((crh, 9 * ch), const),
            pl.BlockSpec((crh, 1), const),
            pl.BlockSpec((ch, crh), const),
            pl.BlockSpec((ch, 1), const),
            pl.BlockSpec((crh, 9 * ch), const),
            pl.BlockSpec((crh, 1), const),
            pl.BlockSpec((ch, crh), const),
            pl.BlockSpec((ch, 1), const),
            pl.BlockSpec((8, cu, 2 * ch), const3),
            pl.BlockSpec((cu, 1), const),
            pl.BlockSpec((16 * co, 16 * cu), const),
            pl.BlockSpec((16 * co, 1), const),
        ],
        out_specs=pl.BlockSpec((16 * co, _IMGS * l_out_p), lambda i: (0, i)),
        scratch_shapes=[
            pltpu.VMEM((_IMGS, 9 * c, l_out_p), _BF),
            pltpu.VMEM((_IMGS, 9 * ch, l_out_p), _BF),
            pltpu.VMEM((_IMGS, 9 * ch, l_out_p), _BF),
            pltpu.VMEM((_IMGS, 9 * ch, l_out_p), _BF),
            pltpu.VMEM((_IMGS, 16 * cu, l_out_p), _BF),
            pltpu.VMEM((_IMGS, ch, l_out_p), _BF),
        ],
        compiler_params=cparams,
        cost_estimate=cost,
    )(xf, cw, cb, *r0, *r1, u0l, u0b, wbig, bout)

    o = out2d.reshape(16, co, n, l_out_p)[..., :l_out]
    o = o.reshape(16, co, n, h, w2)[..., :w]
    o = o.reshape(4, 4, co, n, h, w)
    o = jnp.transpose(o, (3, 2, 4, 0, 5, 1)).reshape(n, co, 4 * h, 4 * w)
    return o.astype(_F32)


# speedup vs baseline: 1.1249x; 1.1249x over previous
from functools import partial

import jax
import jax.numpy as jnp
from jax import lax
from jax.experimental import pallas as pl
from jax.experimental.pallas import tpu as pltpu

_LANE = 128
_BF = jnp.bfloat16
_F32 = jnp.float32
_IMGS = 2

_SRC = ((1, -1), (0, 0), (1, 0), (0, 1))
_READERS = (((0, 3),), ((0, 1), (1, 2), (2, 3)), ((1, 0), (2, 1), (3, 2)),
            ((3, 0),))


def _ru(x, m):
    return (x + m - 1) // m * m


def _geom(h, w):
    w2 = w + 2
    l_out = h * w2
    l_out_p = _ru(max(l_out, _LANE), _LANE)
    l_in_p = _ru(2 * w2 + 2 + l_out_p, _LANE)
    return w2, l_out, l_out_p, l_in_p


def _decoder_kernel(xf, cw, cb, r0w3, r0b3, r0w1, r0b1, r1w3, r1b3, r1w1,
                    r1b1, u0l, u0b, wbig, bout, o_ref, *scratch,
                    c, ch, cu, w, w2, l_out, l_out_p, l_in_p):
    shift = w2 + 1
    tail = l_in_p - shift - l_out_p

    def mask(rows):
        lane = lax.broadcasted_iota(jnp.int32, (rows, l_out_p), 1)
        return (lane % w2 < w) & (lane < l_out)

    m_ch = mask(ch)
    m_cu = mask(cu)

    def padded(y, m):
        yb = jnp.where(m, jnp.maximum(y.astype(_BF), 0), 0)
        rows = yb.shape[0]
        return jnp.concatenate(
            [jnp.zeros((rows, shift), _BF), yb, jnp.zeros((rows, tail), _BF)],
            axis=1)

    def stack9(dst, f, rows):
        for t in range(9):
            off = (t // 3) * w2 + (t % 3)
            dst[t * rows:(t + 1) * rows, :] = f[:, off:off + l_out_p]

    imgs = range(_IMGS)
    sb = [scratch[6 * i:6 * i + 6] for i in imgs]

    for i in imgs:
        stack9(sb[i][0], xf[:, i * l_in_p:(i + 1) * l_in_p], c)
    hh = [jnp.dot(cw[...], sb[i][0][...], preferred_element_type=_F32)
          + cb[...] for i in imgs]

    for ridx, (w3, b3, w1, b1) in enumerate(
            ((r0w3, r0b3, r0w1, r0b1), (r1w3, r1b3, r1w1, r1b1))):
        for i in imgs:
            sb[i][5][...] = hh[i].astype(_BF)
            stack9(sb[i][1 + ridx], padded(hh[i], m_ch), ch)
        t3 = [jnp.maximum(
            (jnp.dot(w3[...], sb[i][1 + ridx][...],
                     preferred_element_type=_F32) + b3[...]).astype(_BF), 0)
            for i in imgs]
        hh = [jnp.dot(w1[...], t3[i], preferred_element_type=_F32) + b1[...]
              + sb[i][5][...].astype(_F32) for i in imgs]

    for i in imgs:
        stack9(sb[i][3], padded(hh[i], m_ch), ch)
    for p in range(4):
        ry, rx = p // 2, p % 2
        r0 = (ry * 3 + rx) * ch
        for i in imgs:
            s3 = sb[i][3]
            acc = jnp.dot(u0l[2 * p], s3[r0:r0 + 2 * ch],
                          preferred_element_type=_F32)
            acc = acc + jnp.dot(u0l[2 * p + 1], s3[r0 + 3 * ch:r0 + 5 * ch],
                                preferred_element_type=_F32)
            fp = padded(acc + u0b[...], m_cu)
            for ya in range(4):
                for xb in range(4):
                    if (_SRC[ya][0] * 2 + _SRC[xb][0]) != p:
                        continue
                    off = (1 + _SRC[ya][1]) * w2 + (1 + _SRC[xb][1])
                    s = ya * 4 + xb
                    sb[i][4][s * cu:(s + 1) * cu, :] = fp[:, off:off + l_out_p]

    for i in imgs:
        o = (jnp.dot(wbig[...], sb[i][4][...], preferred_element_type=_F32)
             + bout[...])
        o_ref[:, i * l_out_p:(i + 1) * l_out_p] = o.astype(_BF)


def kernel(x, conv_w, conv_b, res0_w3, res0_b3, res0_w1, res0_b1,
           res1_w3, res1_b3, res1_w1, res1_b1, up0_w, up0_b, up1_w, up1_b):
    n, c, h, w = x.shape
    ch = conv_w.shape[0]
    crh = res0_w3.shape[0]
    cu = up0_w.shape[1]
    co = up1_w.shape[1]
    w2, l_out, l_out_p, l_in_p = _geom(h, w)

    xc = jnp.transpose(x, (1, 0, 2, 3)).astype(_BF)
    xp = jnp.pad(xc, ((0, 0), (0, 0), (1, 1), (1, 1)))
    xf = xp.reshape(c, n, (h + 2) * (w + 2))
    xf = jnp.pad(xf, ((0, 0), (0, 0), (0, l_in_p - (h + 2) * (w + 2))))
    xf = xf.reshape(c, n * l_in_p)

    cw = jnp.transpose(conv_w, (0, 2, 3, 1)).reshape(ch, 9 * c).astype(_BF)
    cb = conv_b.reshape(ch, 1)

    def res_w(w3, b3, w1, b1):
        return (jnp.transpose(w3, (0, 2, 3, 1)).reshape(crh, 9 * ch).astype(_BF),
                b3.reshape(crh, 1),
                w1.reshape(ch, crh).astype(_BF),
                b1.reshape(ch, 1))

    r0 = res_w(res0_w3, res0_b3, res0_w1, res0_b1)
    r1 = res_w(res1_w3, res1_b3, res1_w1, res1_b1)

    wt0 = jnp.transpose(up0_w, (1, 0, 2, 3))
    u0l = jnp.stack([
        jnp.concatenate([wt0[:, :, 3 - 2 * a - ry, 3 - rx],
                         wt0[:, :, 3 - 2 * a - ry, 1 - rx]], axis=1)
        for ry in (0, 1) for rx in (0, 1) for a in (0, 1)], axis=0).astype(_BF)
    u0b = up0_b.reshape(cu, 1)

    wt1 = jnp.transpose(up1_w, (1, 0, 2, 3))
    zero3 = jnp.zeros((co, cu), _F32)
    rows = []
    for qy in range(4):
        for qx in range(4):
            blocks = []
            for ya in range(4):
                khm = dict(_READERS[ya])
                for xb in range(4):
                    kwm = dict(_READERS[xb])
                    if qy in khm and qx in kwm:
                        blocks.append(wt1[:, :, khm[qy], kwm[qx]])
                    else:
                        blocks.append(zero3)
            rows.append(jnp.concatenate(blocks, axis=1))
    wbig = jnp.concatenate(rows, axis=0).astype(_BF)
    bout = jnp.tile(up1_b, 16).reshape(16 * co, 1)

    cparams = pltpu.CompilerParams(
        dimension_semantics=("parallel",),
        vmem_limit_bytes=64 * 1024 * 1024)
    flops_img = 2 * l_out * (9 * ch * c + 2 * (9 * crh * ch + ch * crh)
                             + 16 * cu * ch + 48 * 16 * cu)
    cost = pl.CostEstimate(flops=n * flops_img, transcendentals=0,
                           bytes_accessed=2 * n * c * l_in_p
                           + 2 * n * 16 * co * l_out_p)

    const = lambda i: (0, 0)
    const3 = lambda i: (0, 0, 0)
    out2d = pl.pallas_call(
        partial(_decoder_kernel, c=c, ch=ch, cu=cu, w=w, w2=w2, l_out=l_out,
                l_out_p=l_out_p, l_in_p=l_in_p),
        out_shape=jax.ShapeDtypeStruct((16 * co, n * l_out_p), _BF),
        grid=(n // _IMGS,),
        in_specs=[
            pl.BlockSpec((c, _IMGS * l_in_p), lambda i: (0, i)),
            pl.BlockSpec((ch, 9 * c), const),
            pl.BlockSpec((ch, 1), const),
            pl.BlockSpec((crh, 9 * ch), const),
            pl.BlockSpec((crh, 1), const),
            pl.BlockSpec((ch, crh), const),
            pl.BlockSpec((ch, 1), const),
            pl.BlockSpec((crh, 9 * ch), const),
            pl.BlockSpec((crh, 1), const),
            pl.BlockSpec((ch, crh), const),
            pl.BlockSpec((ch, 1), const),
            pl.BlockSpec((8, cu, 2 * ch), const3),
            pl.BlockSpec((cu, 1), const),
            pl.BlockSpec((16 * co, 16 * cu), const),
            pl.BlockSpec((16 * co, 1), const),
        ],
        out_specs=pl.BlockSpec((16 * co, _IMGS * l_out_p), lambda i: (0, i)),
        scratch_shapes=[
            pltpu.VMEM(shape, _BF)
            for _ in range(_IMGS)
            for shape in ((9 * c, l_out_p), (9 * ch, l_out_p),
                          (9 * ch, l_out_p), (9 * ch, l_out_p),
                          (16 * cu, l_out_p), (ch, l_out_p))
        ],
        compiler_params=cparams,
        cost_estimate=cost,
    )(xf, cw, cb, *r0, *r1, u0l, u0b, wbig, bout)

    o = out2d.reshape(16, co, n, l_out_p)[..., :l_out]
    o = o.reshape(16, co, n, h, w2)[..., :w]
    o = o.reshape(4, 4, co, n, h, w)
    o = jnp.transpose(o, (3, 2, 4, 0, 5, 1)).reshape(n, co, 4 * h, 4 * w)
    return o.astype(_F32)


# speedup vs baseline: 1.1653x; 1.0359x over previous
from functools import partial

import jax
import jax.numpy as jnp
from jax import lax
from jax.experimental import pallas as pl
from jax.experimental.pallas import tpu as pltpu

_LANE = 128
_BF = jnp.bfloat16
_F32 = jnp.float32
_IMGS = 4

_SRC = ((1, -1), (0, 0), (1, 0), (0, 1))
_READERS = (((0, 3),), ((0, 1), (1, 2), (2, 3)), ((1, 0), (2, 1), (3, 2)),
            ((3, 0),))


def _ru(x, m):
    return (x + m - 1) // m * m


def _geom(h, w):
    w2 = w + 2
    l_out = h * w2
    l_out_p = _ru(max(l_out, _LANE), _LANE)
    l_in_p = _ru(2 * w2 + 2 + l_out_p, _LANE)
    return w2, l_out, l_out_p, l_in_p


def _decoder_kernel(xf, cw, cb, r0w3, r0b3, r0w1, r0b1, r1w3, r1b3, r1w1,
                    r1b1, u0l, u0b, wbig, bout, o_ref, *scratch,
                    c, ch, cu, w, w2, l_out, l_out_p, l_in_p):
    shift = w2 + 1
    tail = l_in_p - shift - l_out_p

    def mask(rows):
        lane = lax.broadcasted_iota(jnp.int32, (rows, l_out_p), 1)
        return (lane % w2 < w) & (lane < l_out)

    m_ch = mask(ch)
    m_cu = mask(cu)

    def padded(y, m):
        yb = jnp.where(m, jnp.maximum(y.astype(_BF), 0), 0)
        rows = yb.shape[0]
        return jnp.concatenate(
            [jnp.zeros((rows, shift), _BF), yb, jnp.zeros((rows, tail), _BF)],
            axis=1)

    def stack9(dst, f, rows):
        for t in range(9):
            off = (t // 3) * w2 + (t % 3)
            dst[t * rows:(t + 1) * rows, :] = f[:, off:off + l_out_p]

    imgs = range(_IMGS)
    sb = [scratch[4 * i:4 * i + 4] for i in imgs]

    for i in imgs:
        stack9(sb[i][0], xf[:, i * l_in_p:(i + 1) * l_in_p], c)
    hh = [jnp.dot(cw[...], sb[i][0][...], preferred_element_type=_F32)
          + cb[...] for i in imgs]

    for w3, b3, w1, b1 in ((r0w3, r0b3, r0w1, r0b1),
                           (r1w3, r1b3, r1w1, r1b1)):
        for i in imgs:
            sb[i][3][...] = hh[i].astype(_BF)
            stack9(sb[i][1], padded(hh[i], m_ch), ch)
        t3 = [jnp.maximum(
            (jnp.dot(w3[...], sb[i][1][...],
                     preferred_element_type=_F32) + b3[...]).astype(_BF), 0)
            for i in imgs]
        hh = [jnp.dot(w1[...], t3[i], preferred_element_type=_F32) + b1[...]
              + sb[i][3][...].astype(_F32) for i in imgs]

    for i in imgs:
        stack9(sb[i][1], padded(hh[i], m_ch), ch)
    for p in range(4):
        ry, rx = p // 2, p % 2
        r0 = (ry * 3 + rx) * ch
        for i in imgs:
            s3 = sb[i][1]
            acc = jnp.dot(u0l[2 * p], s3[r0:r0 + 2 * ch],
                          preferred_element_type=_F32)
            acc = acc + jnp.dot(u0l[2 * p + 1], s3[r0 + 3 * ch:r0 + 5 * ch],
                                preferred_element_type=_F32)
            fp = padded(acc + u0b[...], m_cu)
            for ya in range(4):
                for xb in range(4):
                    if (_SRC[ya][0] * 2 + _SRC[xb][0]) != p:
                        continue
                    off = (1 + _SRC[ya][1]) * w2 + (1 + _SRC[xb][1])
                    s = ya * 4 + xb
                    sb[i][2][s * cu:(s + 1) * cu, :] = fp[:, off:off + l_out_p]

    for i in imgs:
        o = (jnp.dot(wbig[...], sb[i][2][...], preferred_element_type=_F32)
             + bout[...])
        o_ref[:, i * l_out_p:(i + 1) * l_out_p] = o.astype(_BF)


def kernel(x, conv_w, conv_b, res0_w3, res0_b3, res0_w1, res0_b1,
           res1_w3, res1_b3, res1_w1, res1_b1, up0_w, up0_b, up1_w, up1_b):
    n, c, h, w = x.shape
    ch = conv_w.shape[0]
    crh = res0_w3.shape[0]
    cu = up0_w.shape[1]
    co = up1_w.shape[1]
    w2, l_out, l_out_p, l_in_p = _geom(h, w)

    xc = jnp.transpose(x, (1, 0, 2, 3)).astype(_BF)
    xp = jnp.pad(xc, ((0, 0), (0, 0), (1, 1), (1, 1)))
    xf = xp.reshape(c, n, (h + 2) * (w + 2))
    xf = jnp.pad(xf, ((0, 0), (0, 0), (0, l_in_p - (h + 2) * (w + 2))))
    xf = xf.reshape(c, n * l_in_p)

    cw = jnp.transpose(conv_w, (0, 2, 3, 1)).reshape(ch, 9 * c).astype(_BF)
    cb = conv_b.reshape(ch, 1)

    def res_w(w3, b3, w1, b1):
        return (jnp.transpose(w3, (0, 2, 3, 1)).reshape(crh, 9 * ch).astype(_BF),
                b3.reshape(crh, 1),
                w1.reshape(ch, crh).astype(_BF),
                b1.reshape(ch, 1))

    r0 = res_w(res0_w3, res0_b3, res0_w1, res0_b1)
    r1 = res_w(res1_w3, res1_b3, res1_w1, res1_b1)

    wt0 = jnp.transpose(up0_w, (1, 0, 2, 3))
    u0l = jnp.stack([
        jnp.concatenate([wt0[:, :, 3 - 2 * a - ry, 3 - rx],
                         wt0[:, :, 3 - 2 * a - ry, 1 - rx]], axis=1)
        for ry in (0, 1) for rx in (0, 1) for a in (0, 1)], axis=0).astype(_BF)
    u0b = up0_b.reshape(cu, 1)

    wt1 = jnp.transpose(up1_w, (1, 0, 2, 3))
    zero3 = jnp.zeros((co, cu), _F32)
    rows = []
    for qy in range(4):
        for qx in range(4):
            blocks = []
            for ya in range(4):
                khm = dict(_READERS[ya])
                for xb in range(4):
                    kwm = dict(_READERS[xb])
                    if qy in khm and qx in kwm:
                        blocks.append(wt1[:, :, khm[qy], kwm[qx]])
                    else:
                        blocks.append(zero3)
            rows.append(jnp.concatenate(blocks, axis=1))
    wbig = jnp.concatenate(rows, axis=0).astype(_BF)
    bout = jnp.tile(up1_b, 16).reshape(16 * co, 1)

    cparams = pltpu.CompilerParams(
        dimension_semantics=("parallel",),
        vmem_limit_bytes=64 * 1024 * 1024)
    flops_img = 2 * l_out * (9 * ch * c + 2 * (9 * crh * ch + ch * crh)
                             + 16 * cu * ch + 48 * 16 * cu)
    cost = pl.CostEstimate(flops=n * flops_img, transcendentals=0,
                           bytes_accessed=2 * n * c * l_in_p
                           + 2 * n * 16 * co * l_out_p)

    const = lambda i: (0, 0)
    const3 = lambda i: (0, 0, 0)
    out2d = pl.pallas_call(
        partial(_decoder_kernel, c=c, ch=ch, cu=cu, w=w, w2=w2, l_out=l_out,
                l_out_p=l_out_p, l_in_p=l_in_p),
        out_shape=jax.ShapeDtypeStruct((16 * co, n * l_out_p), _BF),
        grid=(n // _IMGS,),
        in_specs=[
            pl.BlockSpec((c, _IMGS * l_in_p), lambda i: (0, i)),
            pl.BlockSpec((ch, 9 * c), const),
            pl.BlockSpec((ch, 1), const),
            pl.BlockSpec((crh, 9 * ch), const),
            pl.BlockSpec((crh, 1), const),
            pl.BlockSpec((ch, crh), const),
            pl.BlockSpec((ch, 1), const),
            pl.BlockSpec((crh, 9 * ch), const),
            pl.BlockSpec((crh, 1), const),
            pl.BlockSpec((ch, crh), const),
            pl.BlockSpec((ch, 1), const),
            pl.BlockSpec((8, cu, 2 * ch), const3),
            pl.BlockSpec((cu, 1), const),
            pl.BlockSpec((16 * co, 16 * cu), const),
            pl.BlockSpec((16 * co, 1), const),
        ],
        out_specs=pl.BlockSpec((16 * co, _IMGS * l_out_p), lambda i: (0, i)),
        scratch_shapes=[
            pltpu.VMEM(shape, _BF)
            for _ in range(_IMGS)
            for shape in ((9 * c, l_out_p), (9 * ch, l_out_p),
                          (16 * cu, l_out_p), (ch, l_out_p))
        ],
        compiler_params=cparams,
        cost_estimate=cost,
    )(xf, cw, cb, *r0, *r1, u0l, u0b, wbig, bout)

    o = out2d.reshape(16, co, n, l_out_p)[..., :l_out]
    o = o.reshape(16, co, n, h, w2)[..., :w]
    o = o.reshape(4, 4, co, n, h, w)
    o = jnp.transpose(o, (3, 2, 4, 0, 5, 1)).reshape(n, co, 4 * h, 4 * w)
    return o.astype(_F32)


# speedup vs baseline: 1.3608x; 1.1678x over previous
from functools import partial

import jax
import jax.numpy as jnp
from jax import lax
from jax.experimental import pallas as pl
from jax.experimental.pallas import tpu as pltpu

_LANE = 128
_BF = jnp.bfloat16
_F32 = jnp.float32
_IMGS = 4

_SRC = ((1, -1), (0, 0), (1, 0), (0, 1))
_READERS = (((0, 3),), ((0, 1), (1, 2), (2, 3)), ((1, 0), (2, 1), (3, 2)),
            ((3, 0),))


def _ru(x, m):
    return (x + m - 1) // m * m


def _geom(h, w):
    w2 = w + 2
    l_out = h * w2
    l_out_p = _ru(max(l_out, _LANE), _LANE)
    l_in_p = _ru(2 * w2 + 2 + l_out_p, _LANE)
    return w2, l_out, l_out_p, l_in_p


def _decoder_kernel(xf, cw, cb, r0w3, r0b3, r0w1, r0b1, r1w3, r1b3, r1w1,
                    r1b1, u0l, u0b, wbig, bout, o_ref, *scratch,
                    c, ch, cu, w, w2, l_out, l_out_p, l_in_p):
    shift = w2 + 1
    tail = l_in_p - shift - l_out_p

    def mask(rows):
        lane = lax.broadcasted_iota(jnp.int32, (rows, l_out_p), 1)
        return (lane % w2 < w) & (lane < l_out)

    m_ch = mask(ch)
    m_cu = mask(cu)

    def relu_mask(y, m):
        return jnp.where(m, jnp.maximum(y.astype(_BF), 0), 0)

    def shifted(yb, off):
        s = off - shift
        rows = yb.shape[0]
        if s < 0:
            return jnp.concatenate(
                [jnp.zeros((rows, -s), _BF), yb[:, :l_out_p + s]], axis=1)
        if s == 0:
            return yb
        return jnp.concatenate(
            [yb[:, s:], jnp.zeros((rows, s), _BF)], axis=1)

    def stack3(dst, yb, rows, g):
        for t in range(3 * g, 3 * g + 3):
            off = (t // 3) * w2 + (t % 3)
            dst[t * rows:(t + 1) * rows, :] = shifted(yb, off)

    imgs = range(_IMGS)
    sb = [scratch[4 * i:4 * i + 4] for i in imgs]

    def conv9(dst, ybs, wref, rows, kblk):
        accs = [None] * _IMGS
        for g in range(3):
            for i in imgs:
                stack3(dst(i), ybs[i], rows, g)
            for i in imgs:
                d = jnp.dot(wref[:, g * kblk:(g + 1) * kblk],
                            dst(i)[3 * g * rows:3 * (g + 1) * rows, :],
                            preferred_element_type=_F32)
                accs[i] = d if accs[i] is None else accs[i] + d
        return accs

    for g in range(3):
        for i in imgs:
            for t in range(3 * g, 3 * g + 3):
                off = i * l_in_p + (t // 3) * w2 + (t % 3)
                sb[i][0][t * c:(t + 1) * c, :] = xf[:, off:off + l_out_p]
    hh = [None] * _IMGS
    for g in range(3):
        for i in imgs:
            d = jnp.dot(cw[:, g * 3 * c:(g + 1) * 3 * c],
                        sb[i][0][3 * g * c:3 * (g + 1) * c, :],
                        preferred_element_type=_F32)
            hh[i] = d if hh[i] is None else hh[i] + d
    hh = [hh[i] + cb[...] for i in imgs]

    for w3, b3, w1, b1 in ((r0w3, r0b3, r0w1, r0b1),
                           (r1w3, r1b3, r1w1, r1b1)):
        ybs = []
        for i in imgs:
            sb[i][3][...] = hh[i].astype(_BF)
            ybs.append(relu_mask(hh[i], m_ch))
        accs = conv9(lambda i: sb[i][1], ybs, w3, ch, 3 * ch)
        t3 = [jnp.maximum((accs[i] + b3[...]).astype(_BF), 0) for i in imgs]
        hh = [jnp.dot(w1[...], t3[i], preferred_element_type=_F32) + b1[...]
              + sb[i][3][...].astype(_F32) for i in imgs]

    ybs = [relu_mask(hh[i], m_ch) for i in imgs]
    for g in range(3):
        for i in imgs:
            stack3(sb[i][1], ybs[i], ch, g)
    for p in range(4):
        ry, rx = p // 2, p % 2
        r0 = (ry * 3 + rx) * ch
        for i in imgs:
            s3 = sb[i][1]
            acc = jnp.dot(u0l[2 * p], s3[r0:r0 + 2 * ch],
                          preferred_element_type=_F32)
            acc = acc + jnp.dot(u0l[2 * p + 1], s3[r0 + 3 * ch:r0 + 5 * ch],
                                preferred_element_type=_F32)
            fp = relu_mask(acc + u0b[...], m_cu)
            for ya in range(4):
                for xb in range(4):
                    if (_SRC[ya][0] * 2 + _SRC[xb][0]) != p:
                        continue
                    off = (1 + _SRC[ya][1]) * w2 + (1 + _SRC[xb][1])
                    s = ya * 4 + xb
                    sb[i][2][s * cu:(s + 1) * cu, :] = shifted(fp, off)

    for i in imgs:
        o = (jnp.dot(wbig[...], sb[i][2][...], preferred_element_type=_F32)
             + bout[...])
        o_ref[:, i * l_out_p:(i + 1) * l_out_p] = o.astype(_BF)


def kernel(x, conv_w, conv_b, res0_w3, res0_b3, res0_w1, res0_b1,
           res1_w3, res1_b3, res1_w1, res1_b1, up0_w, up0_b, up1_w, up1_b):
    n, c, h, w = x.shape
    ch = conv_w.shape[0]
    crh = res0_w3.shape[0]
    cu = up0_w.shape[1]
    co = up1_w.shape[1]
    w2, l_out, l_out_p, l_in_p = _geom(h, w)

    xc = jnp.transpose(x, (1, 0, 2, 3)).astype(_BF)
    xp = jnp.pad(xc, ((0, 0), (0, 0), (1, 1), (1, 1)))
    xf = xp.reshape(c, n, (h + 2) * (w + 2))
    xf = jnp.pad(xf, ((0, 0), (0, 0), (0, l_in_p - (h + 2) * (w + 2))))
    xf = xf.reshape(c, n * l_in_p)

    cw = jnp.transpose(conv_w, (0, 2, 3, 1)).reshape(ch, 9 * c).astype(_BF)
    cb = conv_b.reshape(ch, 1)

    def res_w(w3, b3, w1, b1):
        return (jnp.transpose(w3, (0, 2, 3, 1)).reshape(crh, 9 * ch).astype(_BF),
                b3.reshape(crh, 1),
                w1.reshape(ch, crh).astype(_BF),
                b1.reshape(ch, 1))

    r0 = res_w(res0_w3, res0_b3, res0_w1, res0_b1)
    r1 = res_w(res1_w3, res1_b3, res1_w1, res1_b1)

    wt0 = jnp.transpose(up0_w, (1, 0, 2, 3))
    u0l = jnp.stack([
        jnp.concatenate([wt0[:, :, 3 - 2 * a - ry, 3 - rx],
                         wt0[:, :, 3 - 2 * a - ry, 1 - rx]], axis=1)
        for ry in (0, 1) for rx in (0, 1) for a in (0, 1)], axis=0).astype(_BF)
    u0b = up0_b.reshape(cu, 1)

    wt1 = jnp.transpose(up1_w, (1, 0, 2, 3))
    zero3 = jnp.zeros((co, cu), _F32)
    rows = []
    for qy in range(4):
        for qx in range(4):
            blocks = []
            for ya in range(4):
                khm = dict(_READERS[ya])
                for xb in range(4):
                    kwm = dict(_READERS[xb])
                    if qy in khm and qx in kwm:
                        blocks.append(wt1[:, :, khm[qy], kwm[qx]])
                    else:
                        blocks.append(zero3)
            rows.append(jnp.concatenate(blocks, axis=1))
    wbig = jnp.concatenate(rows, axis=0).astype(_BF)
    bout = jnp.tile(up1_b, 16).reshape(16 * co, 1)

    cparams = pltpu.CompilerParams(
        dimension_semantics=("parallel",),
        vmem_limit_bytes=64 * 1024 * 1024)
    flops_img = 2 * l_out * (9 * ch * c + 2 * (9 * crh * ch + ch * crh)
                             + 16 * cu * ch + 48 * 16 * cu)
    cost = pl.CostEstimate(flops=n * flops_img, transcendentals=0,
                           bytes_accessed=2 * n * c * l_in_p
                           + 2 * n * 16 * co * l_out_p)

    const = lambda i: (0, 0)
    const3 = lambda i: (0, 0, 0)
    step = lambda i: (0, i)
    out2d = pl.pallas_call(
        partial(_decoder_kernel, c=c, ch=ch, cu=cu, w=w, w2=w2, l_out=l_out,
                l_out_p=l_out_p, l_in_p=l_in_p),
        out_shape=jax.ShapeDtypeStruct((16 * co, n * l_out_p), _BF),
        grid=(n // _IMGS,),
        in_specs=[
            pl.BlockSpec((c, _IMGS * l_in_p), step),
            pl.BlockSpec((ch, 9 * c), const),
            pl.BlockSpec((ch, 1), const),
            pl.BlockSpec((crh, 9 * ch), const),
            pl.BlockSpec((crh, 1), const),
            pl.BlockSpec((ch, crh), const),
            pl.BlockSpec((ch, 1), const),
            pl.BlockSpec((crh, 9 * ch), const),
            pl.BlockSpec((crh, 1), const),
            pl.BlockSpec((ch, crh), const),
            pl.BlockSpec((ch, 1), const),
            pl.BlockSpec((8, cu, 2 * ch), const3),
            pl.BlockSpec((cu, 1), const),
            pl.BlockSpec((16 * co, 16 * cu), const),
            pl.BlockSpec((16 * co, 1), const),
        ],
        out_specs=pl.BlockSpec((16 * co, _IMGS * l_out_p), step),
        scratch_shapes=[
            pltpu.VMEM(shape, _BF)
            for _ in range(_IMGS)
            for shape in ((9 * c, l_out_p), (9 * ch, l_out_p),
                          (16 * cu, l_out_p), (ch, l_out_p))
        ],
        compiler_params=cparams,
        cost_estimate=cost,
    )(xf, cw, cb, *r0, *r1, u0l, u0b, wbig, bout)

    o = out2d.reshape(16, co, n, l_out_p)[..., :l_out]
    o = o.reshape(16, co, n, h, w2)[..., :w]
    o = o.reshape(4, 4, co, n, h, w)
    o = jnp.transpose(o, (3, 2, 4, 0, 5, 1)).reshape(n, co, 4 * h, 4 * w)
    return o.astype(_F32)


# speedup vs baseline: 1.3717x; 1.0081x over previous
from functools import partial

import jax
import jax.numpy as jnp
from jax import lax
from jax.experimental import pallas as pl
from jax.experimental.pallas import tpu as pltpu

_LANE = 128
_BF = jnp.bfloat16
_F32 = jnp.float32
_IMGS = 4

_SRC = ((1, -1), (0, 0), (1, 0), (0, 1))
_READERS = (((0, 3),), ((0, 1), (1, 2), (2, 3)), ((1, 0), (2, 1), (3, 2)),
            ((3, 0),))


def _ru(x, m):
    return (x + m - 1) // m * m


def _geom(h, w):
    w2 = w + 2
    l_out = h * w2
    l_out_p = _ru(max(l_out, _LANE), _LANE)
    l_in_p = _ru(2 * w2 + 2 + l_out_p, _LANE)
    return w2, l_out, l_out_p, l_in_p


def _decoder_kernel(xf, cw, cb, r0w3, r0b3, r0w1, r0b1, r1w3, r1b3, r1w1,
                    r1b1, u0l, u0b, wbig, bout, o_ref, *scratch,
                    c, ch, cu, w, w2, l_out, l_out_p, l_in_p):
    shift = w2 + 1
    tail = l_in_p - shift - l_out_p

    def mask(rows):
        lane = lax.broadcasted_iota(jnp.int32, (rows, l_out_p), 1)
        return (lane % w2 < w) & (lane < l_out)

    m_ch = mask(ch)
    m_cu = mask(cu)

    def relu_mask(y, m):
        return jnp.where(m, jnp.maximum(y.astype(_BF), 0), 0)

    def shifted(yb, off):
        s = off - shift
        rows = yb.shape[0]
        if s < 0:
            return jnp.concatenate(
                [jnp.zeros((rows, -s), _BF), yb[:, :l_out_p + s]], axis=1)
        if s == 0:
            return yb
        return jnp.concatenate(
            [yb[:, s:], jnp.zeros((rows, s), _BF)], axis=1)

    def stack3(dst, yb, rows, g):
        for t in range(3 * g, 3 * g + 3):
            off = (t // 3) * w2 + (t % 3)
            dst[t * rows:(t + 1) * rows, :] = shifted(yb, off)

    imgs = range(_IMGS)
    sb = [scratch[4 * i:4 * i + 4] for i in imgs]

    def conv9(dst, ybs, wref, rows, kblk):
        accs = [None] * _IMGS
        for g in range(3):
            for i in imgs:
                stack3(dst(i), ybs[i], rows, g)
            for i in imgs:
                d = jnp.dot(wref[:, g * kblk:(g + 1) * kblk],
                            dst(i)[3 * g * rows:3 * (g + 1) * rows, :],
                            preferred_element_type=_F32)
                accs[i] = d if accs[i] is None else accs[i] + d
        return accs

    for g in range(3):
        for i in imgs:
            for t in range(3 * g, 3 * g + 3):
                off = i * l_in_p + (t // 3) * w2 + (t % 3)
                sb[i][0][t * c:(t + 1) * c, :] = xf[:, off:off + l_out_p]
    hh = [None] * _IMGS
    for g in range(3):
        for i in imgs:
            d = jnp.dot(cw[:, g * 3 * c:(g + 1) * 3 * c],
                        sb[i][0][3 * g * c:3 * (g + 1) * c, :],
                        preferred_element_type=_F32)
            hh[i] = d if hh[i] is None else hh[i] + d
    hh = [hh[i] + cb[...] for i in imgs]

    for w3, b3, w1, b1 in ((r0w3, r0b3, r0w1, r0b1),
                           (r1w3, r1b3, r1w1, r1b1)):
        ybs = []
        for i in imgs:
            sb[i][3][...] = hh[i].astype(_BF)
            ybs.append(relu_mask(hh[i], m_ch))
        accs = conv9(lambda i: sb[i][1], ybs, w3, ch, 3 * ch)
        t3 = [jnp.maximum((accs[i] + b3[...]).astype(_BF), 0) for i in imgs]
        hh = [jnp.dot(w1[...], t3[i], preferred_element_type=_F32) + b1[...]
              + sb[i][3][...].astype(_F32) for i in imgs]

    ybs = [relu_mask(hh[i], m_ch) for i in imgs]
    oacc = [None] * _IMGS

    def up0_phase(p):
        ry, rx = p // 2, p % 2
        r0 = (ry * 3 + rx) * ch
        for i in imgs:
            s3 = sb[i][1]
            acc = jnp.dot(u0l[2 * p], s3[r0:r0 + 2 * ch],
                          preferred_element_type=_F32)
            acc = acc + jnp.dot(u0l[2 * p + 1], s3[r0 + 3 * ch:r0 + 5 * ch],
                                preferred_element_type=_F32)
            fp = relu_mask(acc + u0b[...], m_cu)
            for ya in range(4):
                for xb in range(4):
                    if (_SRC[ya][0] * 2 + _SRC[xb][0]) != p:
                        continue
                    off = (1 + _SRC[ya][1]) * w2 + (1 + _SRC[xb][1])
                    s = ya * 4 + xb
                    sb[i][2][s * cu:(s + 1) * cu, :] = shifted(fp, off)

    def up1_partial(ya):
        for i in imgs:
            d = jnp.dot(wbig[:, ya * 4 * cu:(ya + 1) * 4 * cu],
                        sb[i][2][ya * 4 * cu:(ya + 1) * 4 * cu, :],
                        preferred_element_type=_F32)
            oacc[i] = d if oacc[i] is None else oacc[i] + d

    for g in range(3):
        for i in imgs:
            stack3(sb[i][1], ybs[i], ch, g)
    up0_phase(0)
    up0_phase(1)
    up1_partial(1)
    up1_partial(3)
    up0_phase(2)
    up0_phase(3)
    up1_partial(0)
    up1_partial(2)
    for i in imgs:
        o_ref[:, i * l_out_p:(i + 1) * l_out_p] = (
            oacc[i] + bout[...]).astype(_BF)


def kernel(x, conv_w, conv_b, res0_w3, res0_b3, res0_w1, res0_b1,
           res1_w3, res1_b3, res1_w1, res1_b1, up0_w, up0_b, up1_w, up1_b):
    n, c, h, w = x.shape
    ch = conv_w.shape[0]
    crh = res0_w3.shape[0]
    cu = up0_w.shape[1]
    co = up1_w.shape[1]
    w2, l_out, l_out_p, l_in_p = _geom(h, w)

    xc = jnp.transpose(x, (1, 0, 2, 3)).astype(_BF)
    xp = jnp.pad(xc, ((0, 0), (0, 0), (1, 1), (1, 1)))
    xf = xp.reshape(c, n, (h + 2) * (w + 2))
    xf = jnp.pad(xf, ((0, 0), (0, 0), (0, l_in_p - (h + 2) * (w + 2))))
    xf = xf.reshape(c, n * l_in_p)

    cw = jnp.transpose(conv_w, (0, 2, 3, 1)).reshape(ch, 9 * c).astype(_BF)
    cb = conv_b.reshape(ch, 1)

    def res_w(w3, b3, w1, b1):
        return (jnp.transpose(w3, (0, 2, 3, 1)).reshape(crh, 9 * ch).astype(_BF),
                b3.reshape(crh, 1),
                w1.reshape(ch, crh).astype(_BF),
                b1.reshape(ch, 1))

    r0 = res_w(res0_w3, res0_b3, res0_w1, res0_b1)
    r1 = res_w(res1_w3, res1_b3, res1_w1, res1_b1)

    wt0 = jnp.transpose(up0_w, (1, 0, 2, 3))
    u0l = jnp.stack([
        jnp.concatenate([wt0[:, :, 3 - 2 * a - ry, 3 - rx],
                         wt0[:, :, 3 - 2 * a - ry, 1 - rx]], axis=1)
        for ry in (0, 1) for rx in (0, 1) for a in (0, 1)], axis=0).astype(_BF)
    u0b = up0_b.reshape(cu, 1)

    wt1 = jnp.transpose(up1_w, (1, 0, 2, 3))
    zero3 = jnp.zeros((co, cu), _F32)
    rows = []
    for qy in range(4):
        for qx in range(4):
            blocks = []
            for ya in range(4):
                khm = dict(_READERS[ya])
                for xb in range(4):
                    kwm = dict(_READERS[xb])
                    if qy in khm and qx in kwm:
                        blocks.append(wt1[:, :, khm[qy], kwm[qx]])
                    else:
                        blocks.append(zero3)
            rows.append(jnp.concatenate(blocks, axis=1))
    wbig = jnp.concatenate(rows, axis=0).astype(_BF)
    bout = jnp.tile(up1_b, 16).reshape(16 * co, 1)

    cparams = pltpu.CompilerParams(
        dimension_semantics=("parallel",),
        vmem_limit_bytes=64 * 1024 * 1024)
    flops_img = 2 * l_out * (9 * ch * c + 2 * (9 * crh * ch + ch * crh)
                             + 16 * cu * ch + 48 * 16 * cu)
    cost = pl.CostEstimate(flops=n * flops_img, transcendentals=0,
                           bytes_accessed=2 * n * c * l_in_p
                           + 2 * n * 16 * co * l_out_p)

    const = lambda i: (0, 0)
    const3 = lambda i: (0, 0, 0)
    step = lambda i: (0, i)
    out2d = pl.pallas_call(
        partial(_decoder_kernel, c=c, ch=ch, cu=cu, w=w, w2=w2, l_out=l_out,
                l_out_p=l_out_p, l_in_p=l_in_p),
        out_shape=jax.ShapeDtypeStruct((16 * co, n * l_out_p), _BF),
        grid=(n // _IMGS,),
        in_specs=[
            pl.BlockSpec((c, _IMGS * l_in_p), step),
            pl.BlockSpec((ch, 9 * c), const),
            pl.BlockSpec((ch, 1), const),
            pl.BlockSpec((crh, 9 * ch), const),
            pl.BlockSpec((crh, 1), const),
            pl.BlockSpec((ch, crh), const),
            pl.BlockSpec((ch, 1), const),
            pl.BlockSpec((crh, 9 * ch), const),
            pl.BlockSpec((crh, 1), const),
            pl.BlockSpec((ch, crh), const),
            pl.BlockSpec((ch, 1), const),
            pl.BlockSpec((8, cu, 2 * ch), const3),
            pl.BlockSpec((cu, 1), const),
            pl.BlockSpec((16 * co, 16 * cu), const),
            pl.BlockSpec((16 * co, 1), const),
        ],
        out_specs=pl.BlockSpec((16 * co, _IMGS * l_out_p), step),
        scratch_shapes=[
            pltpu.VMEM(shape, _BF)
            for _ in range(_IMGS)
            for shape in ((9 * c, l_out_p), (9 * ch, l_out_p),
                          (16 * cu, l_out_p), (ch, l_out_p))
        ],
        compiler_params=cparams,
        cost_estimate=cost,
    )(xf, cw, cb, *r0, *r1, u0l, u0b, wbig, bout)

    o = out2d.reshape(16, co, n, l_out_p)[..., :l_out]
    o = o.reshape(16, co, n, h, w2)[..., :w]
    o = o.reshape(4, 4, co, n, h, w)
    o = jnp.transpose(o, (3, 2, 4, 0, 5, 1)).reshape(n, co, 4 * h, 4 * w)
    return o.astype(_F32)
